# baseline (speedup 1.0000x reference)
import functools

import jax
import jax.numpy as jnp
from jax import lax
from jax.experimental import pallas as pl
from jax.experimental.pallas import tpu as pltpu
from jax.experimental.pallas import tpu_sc as plsc


def _const_embed_sc(pos_embed, batch_n):
    S, D = pos_embed.shape
    NC, NS = 2, 16
    NW = NC * NS
    rows_per_w = S // NW

    mesh = plsc.VectorSubcoreMesh(core_axis_name="c", subcore_axis_name="s")

    @functools.partial(
        pl.kernel,
        out_type=jax.ShapeDtypeStruct((S, batch_n, D), jnp.float32),
        mesh=mesh,
        scratch_types=[
            pltpu.VMEM((rows_per_w, D), jnp.float32),
            pltpu.SemaphoreType.DMA,
            pltpu.SemaphoreType.DMA,
        ],
    )
    def k(pe_hbm, out_hbm, rows_v, gsem, ssem):
        wid = lax.axis_index("c") * NS + lax.axis_index("s")
        base = wid * rows_per_w
        splits = [(0, rows_per_w // 4), (rows_per_w // 4, 3 * rows_per_w // 4)]
        gathers = [
            pltpu.make_async_copy(
                pe_hbm.at[pl.ds(base + off, ln)],
                rows_v.at[pl.ds(off, ln)],
                gsem,
            )
            for off, ln in splits
        ]
        scatters = [
            pltpu.make_async_copy(
                rows_v.at[pl.ds(off, ln)],
                out_hbm.at[pl.ds(base + off, ln), n],
                ssem,
            )
            for off, ln in splits
            for n in range(batch_n)
        ]
        gathers[0].start()
        gathers[1].start()
        gathers[0].wait()
        for c in scatters[:batch_n]:
            c.start()
        gathers[1].wait()
        for c in scatters[batch_n:]:
            c.start()
        for c in scatters:
            c.wait()

    return k(pos_embed)


def kernel(z, pos_embed):
    return _const_embed_sc(pos_embed, z.shape[1])


# speedup vs baseline: 1.0240x; 1.0240x over previous
import functools

import jax
import jax.numpy as jnp
from jax import lax
from jax.experimental import pallas as pl
from jax.experimental.pallas import tpu as pltpu
from jax.experimental.pallas import tpu_sc as plsc


def _const_embed_sc(pos_embed, batch_n):
    S, D = pos_embed.shape
    NC, NS = 2, 16
    NW = NC * NS
    rows_per_w = S // NW

    mesh = plsc.VectorSubcoreMesh(core_axis_name="c", subcore_axis_name="s")

    @functools.partial(
        pl.kernel,
        out_type=jax.ShapeDtypeStruct((S, batch_n, D), jnp.float32),
        mesh=mesh,
        scratch_types=[
            pltpu.VMEM((rows_per_w, D), jnp.float32),
            pltpu.SemaphoreType.DMA,
            pltpu.SemaphoreType.DMA,
        ],
    )
    def k(pe_hbm, out_hbm, rows_v, gsem, ssem):
        wid = lax.axis_index("c") * NS + lax.axis_index("s")
        base = wid * rows_per_w
        half = rows_per_w // 2
        gathers = [
            pltpu.make_async_copy(
                pe_hbm.at[pl.ds(base + j * half, half)],
                rows_v.at[pl.ds(j * half, half)],
                gsem,
            )
            for j in range(2)
        ]
        scatters = [
            pltpu.make_async_copy(
                rows_v.at[pl.ds(j * half, half)],
                out_hbm.at[pl.ds(base + j * half, half), n],
                ssem,
            )
            for j in range(2)
            for n in range(batch_n)
        ]
        gathers[0].start()
        gathers[1].start()
        gathers[0].wait()
        for c in scatters[:batch_n]:
            c.start()
        gathers[1].wait()
        for c in scatters[batch_n:]:
            c.start()
        for c in scatters:
            c.wait()

    return k(pos_embed)


def kernel(z, pos_embed):
    return _const_embed_sc(pos_embed, z.shape[1])
